# baseline (speedup 1.0000x reference)
import jax
import jax.numpy as jnp
from jax import lax
from jax.experimental import pallas as pl
from jax.experimental.pallas import tpu as pltpu
from jax.experimental.pallas import tpu_sc as plsc


def _build_sc_kernel(B, S, V, H, C, NBUF, LOOK, HALF):
    info = plsc.get_sparse_core_info()
    NC, NS, L = info.num_cores, info.num_subcores, info.num_lanes
    NW = NC * NS
    SW = S // NW
    assert S % NW == 0 and SW % HALF == 0 and HALF % C == 0 and H % L == 0
    cph = HALF // C
    steps_per_half = B * cph
    nsteps = (SW // HALF) * steps_per_half

    mesh = plsc.VectorSubcoreMesh(core_axis_name="c", subcore_axis_name="s")

    import functools

    @functools.partial(
        pl.kernel,
        mesh=mesh,
        out_type=jax.ShapeDtypeStruct((B, S, H), jnp.float32),
        scratch_types=[
            pltpu.VMEM((B, SW), jnp.int32),
            pltpu.VMEM((HALF, H), jnp.float32),
            pltpu.VMEM((NBUF, C, H), jnp.float32),
            pltpu.SemaphoreType.DMA,
            pltpu.SemaphoreType.DMA,
        ],
    )
    def k(ids_hbm, tok_hbm, pos_hbm, out_hbm, idx_v, pos_v, tok_v, gsem, ssem):
        wid = lax.axis_index("s") * NC + lax.axis_index("c")
        s_base = wid * SW
        idx_cps = [pltpu.async_copy(ids_hbm.at[b, pl.ds(s_base, SW)],
                                    idx_v.at[b], gsem) for b in range(B)]
        for cp in idx_cps:
            cp.wait()

        def coords(t):
            j, r = divmod(t, steps_per_half)
            b, jj = divmod(r, cph)
            return j, b, jj

        def gather_desc(t):
            j, b, jj = coords(t)
            off = j * HALF + jj * C
            return pltpu.make_async_copy(
                tok_hbm.at[idx_v.at[b, pl.ds(off, C)]],
                tok_v.at[t % NBUF], gsem)

        def store_desc(t):
            j, b, jj = coords(t)
            return pltpu.make_async_copy(
                tok_v.at[t % NBUF],
                out_hbm.at[b, pl.ds(s_base + j * HALF + jj * C, C)],
                ssem)

        for t in range(LOOK):
            gather_desc(t).start()

        def body(t, carry):
            nxt = t + LOOK

            @pl.when(jnp.logical_and(nxt >= NBUF, nxt < nsteps))
            def _():
                store_desc(nxt - NBUF).wait()

            @pl.when(nxt < nsteps)
            def _():
                gather_desc(nxt).start()

            @pl.when(t % steps_per_half == 0)
            def _():
                j = t // steps_per_half
                pltpu.sync_copy(pos_hbm.at[pl.ds(s_base + j * HALF, HALF)],
                                pos_v)

            gather_desc(t).wait()
            buf = t % NBUF
            _, _, jj = coords(t)
            p0 = jj * C

            @plsc.parallel_loop(0, C, unroll=2)
            def _add(r):
                for kk in range(H // L):
                    sl = pl.ds(kk * L, L)
                    plsc.addupdate(tok_v.at[buf, r, sl], pos_v[p0 + r, sl])

            store_desc(t).start()
            return carry

        lax.fori_loop(0, nsteps, body, 0)
        for t in range(nsteps - NBUF, nsteps):
            store_desc(t).wait()

    return k


def kernel(input_ids, tok_table, pos_table):
    B, S = input_ids.shape
    V, H = tok_table.shape
    k = _build_sc_kernel(B, S, V, H, C=32, NBUF=2, LOOK=1, HALF=32)
    return k(input_ids.astype(jnp.int32), tok_table, pos_table)


# speedup vs baseline: 1.1691x; 1.1691x over previous
import jax
import jax.numpy as jnp
from jax import lax
from jax.experimental import pallas as pl
from jax.experimental.pallas import tpu as pltpu
from jax.experimental.pallas import tpu_sc as plsc


def _build_sc_kernel(B, S, V, H, C, NBUF, LOOK, HALF):
    info = plsc.get_sparse_core_info()
    NC, NS, L = info.num_cores, info.num_subcores, info.num_lanes
    NW = NC * NS
    SW = S // NW
    assert S % NW == 0 and SW % HALF == 0 and HALF % C == 0 and H % L == 0
    cph = HALF // C
    steps_per_half = B * cph
    nsteps = (SW // HALF) * steps_per_half

    mesh = plsc.VectorSubcoreMesh(core_axis_name="c", subcore_axis_name="s")

    import functools

    @functools.partial(
        pl.kernel,
        mesh=mesh,
        out_type=jax.ShapeDtypeStruct((B, S, H), jnp.float32),
        scratch_types=[
            pltpu.VMEM((B, SW), jnp.int32),
            pltpu.VMEM((HALF, H), jnp.float32),
            pltpu.VMEM((NBUF, C, H), jnp.float32),
            pltpu.SemaphoreType.DMA,
            pltpu.SemaphoreType.DMA,
            pltpu.SemaphoreType.DMA,
        ],
    )
    def k(ids_hbm, tok_hbm, pos_hbm, out_hbm, idx_v, pos_v, tok_v,
          gsem, ssem, psem):
        wid = lax.axis_index("s") * NC + lax.axis_index("c")
        s_base = wid * SW

        def pos_desc(j):
            return pltpu.make_async_copy(
                pos_hbm.at[pl.ds(s_base + j * HALF, HALF)], pos_v, psem)

        pos_desc(0).start()
        idx_cps = [pltpu.async_copy(ids_hbm.at[b, pl.ds(s_base, SW)],
                                    idx_v.at[b], gsem) for b in range(B)]
        for cp in idx_cps:
            cp.wait()

        def coords(t):
            j, r = divmod(t, steps_per_half)
            b, jj = divmod(r, cph)
            return j, b, jj

        def gather_desc(t):
            j, b, jj = coords(t)
            off = j * HALF + jj * C
            return pltpu.make_async_copy(
                tok_hbm.at[idx_v.at[b, pl.ds(off, C)]],
                tok_v.at[t % NBUF], gsem)

        def store_desc(t):
            j, b, jj = coords(t)
            return pltpu.make_async_copy(
                tok_v.at[t % NBUF],
                out_hbm.at[b, pl.ds(s_base + j * HALF + jj * C, C)],
                ssem)

        for t in range(LOOK):
            gather_desc(t).start()

        def body(t, carry):
            nxt = t + LOOK

            @pl.when(jnp.logical_and(nxt >= NBUF, nxt < nsteps))
            def _():
                store_desc(nxt - NBUF).wait()

            @pl.when(nxt < nsteps)
            def _():
                gather_desc(nxt).start()

            @pl.when(t % steps_per_half == 0)
            def _():
                pos_desc(t // steps_per_half).wait()

            gather_desc(t).wait()
            buf = t % NBUF
            _, _, jj = coords(t)
            p0 = jj * C

            @plsc.parallel_loop(0, C, unroll=2)
            def _add(r):
                for kk in range(H // L):
                    sl = pl.ds(kk * L, L)
                    plsc.addupdate(tok_v.at[buf, r, sl], pos_v[p0 + r, sl])

            store_desc(t).start()

            @pl.when(jnp.logical_and(t % steps_per_half == steps_per_half - 1,
                                     t + 1 < nsteps))
            def _():
                pos_desc(t // steps_per_half + 1).start()

            return carry

        lax.fori_loop(0, nsteps, body, 0)
        for t in range(nsteps - NBUF, nsteps):
            store_desc(t).wait()

    return k


def kernel(input_ids, tok_table, pos_table):
    B, S = input_ids.shape
    V, H = tok_table.shape
    k = _build_sc_kernel(B, S, V, H, C=16, NBUF=5, LOOK=3, HALF=32)
    return k(input_ids.astype(jnp.int32), tok_table, pos_table)
